# baseline (speedup 1.0000x reference)
import functools

import jax
import jax.numpy as jnp
import numpy as np
from jax import lax
from jax.experimental import pallas as pl
from jax.experimental.pallas import tpu as pltpu
from jax.experimental.pallas import tpu_sc as plsc

B, N, C = 4, 8192, 768
HID = 128
NUM_KEEP = 5734

NC, NS = 2, 16
NW = NC * NS
KPAD = 5760
PER_W = (B * KPAD) // NW
GCHUNK = 48
NGC = PER_W // GCHUNK
SCHUNK = PER_W // 16

_MININT = np.int32(-2**31)
_BITVALS = [np.int32(np.uint32(1 << i)) for i in range(31, -1, -1)]


def _scores_body(tok_ref, w1t_ref, b1_ref, w2t_ref, b2_ref, out_ref):
    x = tok_ref[...]
    h = jnp.dot(x, w1t_ref[...], preferred_element_type=jnp.float32)
    h = h + b1_ref[...]
    h = 0.5 * h * (1.0 + lax.erf(h * np.float32(0.7071067811865476)))
    s = jnp.dot(h, w2t_ref[...], preferred_element_type=jnp.float32)
    out_ref[...] = s + b2_ref[...]


def _scores(tokens_flat, W1, b1, W2, b2):
    blk = 1024
    grid = (tokens_flat.shape[0] // blk,)
    return pl.pallas_call(
        _scores_body,
        grid=grid,
        in_specs=[
            pl.BlockSpec((blk, C), lambda i: (i, 0)),
            pl.BlockSpec((C, HID), lambda i: (0, 0)),
            pl.BlockSpec((1, HID), lambda i: (0, 0)),
            pl.BlockSpec((HID, 1), lambda i: (0, 0)),
            pl.BlockSpec((1, 1), lambda i: (0, 0)),
        ],
        out_specs=pl.BlockSpec((blk, 1), lambda i: (i, 0)),
        out_shape=jax.ShapeDtypeStruct((tokens_flat.shape[0], 1), jnp.float32),
    )(tokens_flat, W1.T, b1.reshape(1, HID), W2.T, b2.reshape(1, 1))


def _cumsum_2d(m):
    hi = lax.broadcasted_iota(jnp.int32, (128, 128), 0)
    wi = lax.broadcasted_iota(jnp.int32, (128, 128), 1)
    upper = (hi <= wi).astype(jnp.float32)
    incl_row = jnp.dot(m, upper, precision=lax.Precision.HIGHEST,
                       preferred_element_type=jnp.float32)
    rowtot = jnp.broadcast_to(incl_row[:, 127:128], (64, 128))
    ri = lax.broadcasted_iota(jnp.int32, (64, 64), 0)
    ci = lax.broadcasted_iota(jnp.int32, (64, 64), 1)
    strict = (ci < ri).astype(jnp.float32)
    rowoff = jnp.dot(strict, rowtot, precision=lax.Precision.HIGHEST,
                     preferred_element_type=jnp.float32)
    return incl_row + rowoff


def _select_body(s_ref, cum_ref):
    s = s_ref[...]
    bits = lax.bitcast_convert_type(s, jnp.int32)
    key = bits ^ (lax.shift_right_arithmetic(bits, 31) & np.int32(0x7FFFFFFF))
    for b in range(B):
        w = key[b * 64:(b + 1) * 64, :]
        t_u = jnp.int32(0)
        for bv in _BITVALS:
            cand_u = t_u | bv
            cand_s = cand_u ^ _MININT
            cnt = jnp.sum((w >= cand_s).astype(jnp.int32))
            t_u = jnp.where(cnt >= NUM_KEEP, cand_u, t_u)
        t_s = t_u ^ _MININT
        gt = (w > t_s)
        eq = (w == t_s)
        cnt_gt = jnp.sum(gt.astype(jnp.int32))
        r = (NUM_KEEP - cnt_gt).astype(jnp.float32)
        eq_f = eq.astype(jnp.float32)
        tie_exc = _cumsum_2d(eq_f) - eq_f
        mask = jnp.logical_or(gt, jnp.logical_and(eq, tie_exc < r))
        cum = _cumsum_2d(mask.astype(jnp.float32))
        cum_ref[b * 64:(b + 1) * 64, :] = cum.astype(jnp.int32)


def _select(scores_2d):
    return pl.pallas_call(
        _select_body,
        in_specs=[pl.BlockSpec((256, 128), lambda: (0, 0))],
        out_specs=pl.BlockSpec((256, 128), lambda: (0, 0)),
        out_shape=jax.ShapeDtypeStruct((256, 128), jnp.int32),
    )(scores_2d)


def _sc_body(cum_hbm, tok_hbm, rows_out, idx_out,
             cum_v, idx_v, gidx_v, rows_v, sem):
    wid = lax.axis_index("s") * NC + lax.axis_index("c")
    b = wid // 8
    j = wid % 8
    pbase = j * PER_W
    rbase = jnp.where(j == 7, NUM_KEEP - PER_W, pbase)
    pltpu.sync_copy(cum_hbm.at[pl.ds(pl.multiple_of(b * N, 8), N)], cum_v)

    def make_search(base, out_ref, off):
        def search(ci, _):
            target = base + ci * 16 + lax.iota(jnp.int32, 16) + 1
            lb = jnp.zeros((16,), jnp.int32)
            for step in (4096, 2048, 1024, 512, 256, 128, 64, 32, 16, 8, 4,
                         2, 1):
                probe = lb + (step - 1)
                v = plsc.load_gather(cum_v, [probe])
                lb = lb + jnp.where(v < target, step, 0)
            out_ref[pl.ds(ci * 16, 16)] = lb + off
            return 0
        return search

    lax.fori_loop(0, SCHUNK, make_search(pbase, idx_v, 0), 0)
    pltpu.sync_copy(idx_v, idx_out.at[pl.ds(pl.multiple_of(wid * PER_W, 8),
                                            PER_W)])

    @pl.when(j < 7)
    def _():
        def cp(ci, _):
            gidx_v[pl.ds(ci * 16, 16)] = idx_v[pl.ds(ci * 16, 16)] + b * N
            return 0
        lax.fori_loop(0, SCHUNK, cp, 0)

    @pl.when(j == 7)
    def _():
        lax.fori_loop(0, SCHUNK,
                      make_search(NUM_KEEP - PER_W, gidx_v, b * N), 0)

    def gather(c, _):
        idx_slice = gidx_v.at[pl.ds(c * GCHUNK, GCHUNK)]
        pltpu.async_copy(tok_hbm.at[idx_slice], rows_v, sem).wait()
        pltpu.sync_copy(
            rows_v,
            rows_out.at[pl.ds(b * NUM_KEEP + rbase + c * GCHUNK, GCHUNK)])
        return 0

    lax.fori_loop(0, NGC, gather, 0)


def _sc_gather(cum_flat, tokens_flat):
    mesh = plsc.VectorSubcoreMesh(core_axis_name="c", subcore_axis_name="s",
                                  num_cores=NC, num_subcores=NS)
    f = pl.kernel(
        _sc_body,
        out_type=(
            jax.ShapeDtypeStruct((B * NUM_KEEP, C), jnp.float32),
            jax.ShapeDtypeStruct((B * KPAD,), jnp.int32),
        ),
        mesh=mesh,
        compiler_params=pltpu.CompilerParams(needs_layout_passes=False,
                                             use_tc_tiling_on_sc=False),
        scratch_types=[
            pltpu.VMEM((N,), jnp.int32),
            pltpu.VMEM((PER_W,), jnp.int32),
            pltpu.VMEM((PER_W,), jnp.int32),
            pltpu.VMEM((GCHUNK, C), jnp.float32),
            pltpu.SemaphoreType.DMA,
        ],
    )
    return f(cum_flat, tokens_flat)


def kernel(tokens, spatial_shape, W1, b1, W2, b2):
    tokens_flat = tokens.reshape(B * N, C)
    scores = _scores(tokens_flat, W1, b1, W2, b2)
    cum = _select(scores.reshape(256, 128))
    rows, idx = _sc_gather(cum.reshape(B * N), tokens_flat)
    pruned = rows.reshape(B, NUM_KEEP, C)
    top_idx = idx.reshape(B, KPAD)[:, :NUM_KEEP]
    return pruned, top_idx


# speedup vs baseline: 1.0598x; 1.0598x over previous
import functools

import jax
import jax.numpy as jnp
import numpy as np
from jax import lax
from jax.experimental import pallas as pl
from jax.experimental.pallas import tpu as pltpu
from jax.experimental.pallas import tpu_sc as plsc

B, N, C = 4, 8192, 768
HID = 128
NUM_KEEP = 5734

NC, NS = 2, 16
NW = NC * NS
KPAD = 5760
PER_W = (B * KPAD) // NW
GCHUNK = 48
NGC = PER_W // GCHUNK
SCHUNK = PER_W // 16

_MININT = np.int32(-2**31)
_BITVALS = [np.int32(np.uint32(1 << i)) for i in range(31, -1, -1)]


def _scores_body(tok_ref, w1t_ref, b1_ref, w2t_ref, b2_ref, out_ref):
    x = tok_ref[...]
    h = jnp.dot(x, w1t_ref[...], preferred_element_type=jnp.float32)
    h = h + b1_ref[...]
    h = 0.5 * h * (1.0 + lax.erf(h * np.float32(0.7071067811865476)))
    s = jnp.dot(h, w2t_ref[...], preferred_element_type=jnp.float32)
    out_ref[...] = s + b2_ref[...]


def _scores(tokens_flat, W1, b1, W2, b2):
    blk = 1024
    grid = (tokens_flat.shape[0] // blk,)
    return pl.pallas_call(
        _scores_body,
        grid=grid,
        in_specs=[
            pl.BlockSpec((blk, C), lambda i: (i, 0)),
            pl.BlockSpec((C, HID), lambda i: (0, 0)),
            pl.BlockSpec((1, HID), lambda i: (0, 0)),
            pl.BlockSpec((HID, 1), lambda i: (0, 0)),
            pl.BlockSpec((1, 1), lambda i: (0, 0)),
        ],
        out_specs=pl.BlockSpec((blk, 1), lambda i: (i, 0)),
        out_shape=jax.ShapeDtypeStruct((tokens_flat.shape[0], 1), jnp.float32),
    )(tokens_flat, W1.T, b1.reshape(1, HID), W2.T, b2.reshape(1, 1))


def _cumsum_2d(m):
    hi = lax.broadcasted_iota(jnp.int32, (128, 128), 0)
    wi = lax.broadcasted_iota(jnp.int32, (128, 128), 1)
    upper = (hi <= wi).astype(jnp.float32)
    incl_row = jnp.dot(m, upper, precision=lax.Precision.HIGHEST,
                       preferred_element_type=jnp.float32)
    rowtot = jnp.broadcast_to(incl_row[:, 127:128], (64, 128))
    ri = lax.broadcasted_iota(jnp.int32, (64, 64), 0)
    ci = lax.broadcasted_iota(jnp.int32, (64, 64), 1)
    strict = (ci < ri).astype(jnp.float32)
    rowoff = jnp.dot(strict, rowtot, precision=lax.Precision.HIGHEST,
                     preferred_element_type=jnp.float32)
    return incl_row + rowoff


def _select_body(s_ref, cum_ref):
    s = s_ref[...]
    bits = lax.bitcast_convert_type(s, jnp.int32)
    key = bits ^ (lax.shift_right_arithmetic(bits, 31) & np.int32(0x7FFFFFFF))
    for b in range(B):
        w = key[b * 64:(b + 1) * 64, :]
        t_u = jnp.int32(0)
        for bv in _BITVALS:
            cand_u = t_u | bv
            cand_s = cand_u ^ _MININT
            cnt = jnp.sum((w >= cand_s).astype(jnp.int32))
            t_u = jnp.where(cnt >= NUM_KEEP, cand_u, t_u)
        t_s = t_u ^ _MININT
        gt = (w > t_s)
        eq = (w == t_s)
        cnt_gt = jnp.sum(gt.astype(jnp.int32))
        r = (NUM_KEEP - cnt_gt).astype(jnp.float32)
        eq_f = eq.astype(jnp.float32)
        tie_exc = _cumsum_2d(eq_f) - eq_f
        mask = jnp.logical_or(gt, jnp.logical_and(eq, tie_exc < r))
        cum = _cumsum_2d(mask.astype(jnp.float32))
        cum_ref[b * 64:(b + 1) * 64, :] = cum.astype(jnp.int32)


def _select(scores_2d):
    return pl.pallas_call(
        _select_body,
        in_specs=[pl.BlockSpec((256, 128), lambda: (0, 0))],
        out_specs=pl.BlockSpec((256, 128), lambda: (0, 0)),
        out_shape=jax.ShapeDtypeStruct((256, 128), jnp.int32),
    )(scores_2d)


def _sc_body(cum_hbm, tok_hbm, rows_out, idx_out,
             cum_v, idx_v, gidx_v, oidx_v, rows_v, sem):
    wid = lax.axis_index("s") * NC + lax.axis_index("c")
    b = wid // 8
    j = wid % 8
    pbase = j * PER_W
    rbase = jnp.where(j == 7, NUM_KEEP - PER_W, pbase)
    pltpu.sync_copy(cum_hbm.at[pl.ds(pl.multiple_of(b * N, 8), N)], cum_v)

    def make_search(base, out_ref, off):
        def search(ci, _):
            target = base + ci * 16 + lax.iota(jnp.int32, 16) + 1
            lb = jnp.zeros((16,), jnp.int32)
            for step in (4096, 2048, 1024, 512, 256, 128, 64, 32, 16, 8, 4,
                         2, 1):
                probe = lb + (step - 1)
                v = plsc.load_gather(cum_v, [probe])
                lb = lb + jnp.where(v < target, step, 0)
            out_ref[pl.ds(ci * 16, 16)] = lb + off
            return 0
        return search

    lax.fori_loop(0, SCHUNK, make_search(pbase, idx_v, 0), 0)
    pltpu.sync_copy(idx_v, idx_out.at[pl.ds(pl.multiple_of(wid * PER_W, 8),
                                            PER_W)])

    @pl.when(j < 7)
    def _():
        def cp(ci, _):
            gidx_v[pl.ds(ci * 16, 16)] = idx_v[pl.ds(ci * 16, 16)] + b * N
            return 0
        lax.fori_loop(0, SCHUNK, cp, 0)

    @pl.when(j == 7)
    def _():
        lax.fori_loop(0, SCHUNK,
                      make_search(NUM_KEEP - PER_W, gidx_v, b * N), 0)

    dbase = b * NUM_KEEP + rbase

    def gather(c, _):
        for t in range(GCHUNK // 16):
            oidx_v[c, pl.ds(t * 16, 16)] = (
                dbase + c * GCHUNK + t * 16 + lax.iota(jnp.int32, 16))
        idx_slice = gidx_v.at[pl.ds(c * GCHUNK, GCHUNK)]
        pltpu.async_copy(tok_hbm.at[idx_slice], rows_v, sem).wait()
        pltpu.async_copy(rows_v, rows_out.at[oidx_v.at[c]], sem).wait()
        return 0

    lax.fori_loop(0, NGC, gather, 0)


def _sc_gather(cum_flat, tokens_flat):
    mesh = plsc.VectorSubcoreMesh(core_axis_name="c", subcore_axis_name="s",
                                  num_cores=NC, num_subcores=NS)
    f = pl.kernel(
        _sc_body,
        out_type=(
            jax.ShapeDtypeStruct((B * NUM_KEEP, C), jnp.float32),
            jax.ShapeDtypeStruct((B * KPAD,), jnp.int32),
        ),
        mesh=mesh,
        compiler_params=pltpu.CompilerParams(needs_layout_passes=False),
        scratch_types=[
            pltpu.VMEM((N,), jnp.int32),
            pltpu.VMEM((PER_W,), jnp.int32),
            pltpu.VMEM((PER_W,), jnp.int32),
            pltpu.VMEM((NGC, GCHUNK), jnp.int32),
            pltpu.VMEM((GCHUNK, C), jnp.float32),
            pltpu.SemaphoreType.DMA,
        ],
    )
    return f(cum_flat, tokens_flat)


def kernel(tokens, spatial_shape, W1, b1, W2, b2):
    tokens_flat = tokens.reshape(B * N, C)
    scores = _scores(tokens_flat, W1, b1, W2, b2)
    cum = _select(scores.reshape(256, 128))
    rows, idx = _sc_gather(cum.reshape(B * N), tokens_flat)
    pruned = rows.reshape(B, NUM_KEEP, C)
    top_idx = idx.reshape(B, KPAD)[:, :NUM_KEEP]
    return pruned, top_idx


# speedup vs baseline: 2.5723x; 2.4272x over previous
import functools

import jax
import jax.numpy as jnp
import numpy as np
from jax import lax
from jax.experimental import pallas as pl
from jax.experimental.pallas import tpu as pltpu
from jax.experimental.pallas import tpu_sc as plsc

B, N, C = 4, 8192, 768
HID = 128
NUM_KEEP = 5734

NC, NS = 2, 16
NW = NC * NS
KPAD = 5760
PER_W = (B * KPAD) // NW
GCHUNK = 48
NGC = PER_W // GCHUNK
SCHUNK = PER_W // 16

_MININT = np.int32(-2**31)
_BITVALS = [np.int32(np.uint32(1 << i)) for i in range(31, -1, -1)]


def _scores_body(tok_ref, w1t_ref, b1_ref, w2t_ref, b2_ref, out_ref):
    x = tok_ref[...]
    h = jnp.dot(x, w1t_ref[...], preferred_element_type=jnp.float32)
    h = h + b1_ref[...]
    h = 0.5 * h * (1.0 + lax.erf(h * np.float32(0.7071067811865476)))
    s = jnp.dot(h, w2t_ref[...], preferred_element_type=jnp.float32)
    out_ref[...] = s + b2_ref[...]


def _scores(tokens_flat, W1, b1, W2, b2):
    blk = 1024
    grid = (tokens_flat.shape[0] // blk,)
    return pl.pallas_call(
        _scores_body,
        grid=grid,
        in_specs=[
            pl.BlockSpec((blk, C), lambda i: (i, 0)),
            pl.BlockSpec((C, HID), lambda i: (0, 0)),
            pl.BlockSpec((1, HID), lambda i: (0, 0)),
            pl.BlockSpec((HID, 1), lambda i: (0, 0)),
            pl.BlockSpec((1, 1), lambda i: (0, 0)),
        ],
        out_specs=pl.BlockSpec((blk, 1), lambda i: (i, 0)),
        out_shape=jax.ShapeDtypeStruct((tokens_flat.shape[0], 1), jnp.float32),
    )(tokens_flat, W1.T, b1.reshape(1, HID), W2.T, b2.reshape(1, 1))


def _cumsum_2d(m):
    hi = lax.broadcasted_iota(jnp.int32, (128, 128), 0)
    wi = lax.broadcasted_iota(jnp.int32, (128, 128), 1)
    upper = (hi <= wi).astype(jnp.float32)
    incl_row = jnp.dot(m, upper, precision=lax.Precision.HIGHEST,
                       preferred_element_type=jnp.float32)
    rowtot = jnp.broadcast_to(incl_row[:, 127:128], (64, 128))
    ri = lax.broadcasted_iota(jnp.int32, (64, 64), 0)
    ci = lax.broadcasted_iota(jnp.int32, (64, 64), 1)
    strict = (ci < ri).astype(jnp.float32)
    rowoff = jnp.dot(strict, rowtot, precision=lax.Precision.HIGHEST,
                     preferred_element_type=jnp.float32)
    return incl_row + rowoff


def _select_body(s_ref, cum_ref):
    s = s_ref[...]
    bits = lax.bitcast_convert_type(s, jnp.int32)
    key = bits ^ (lax.shift_right_arithmetic(bits, 31) & np.int32(0x7FFFFFFF))
    for b in range(B):
        w = key[b * 64:(b + 1) * 64, :]
        t_u = jnp.int32(0)
        for bv in _BITVALS:
            cand_u = t_u | bv
            cand_s = cand_u ^ _MININT
            cnt = jnp.sum((w >= cand_s).astype(jnp.int32))
            t_u = jnp.where(cnt >= NUM_KEEP, cand_u, t_u)
        t_s = t_u ^ _MININT
        gt = (w > t_s)
        eq = (w == t_s)
        cnt_gt = jnp.sum(gt.astype(jnp.int32))
        r = (NUM_KEEP - cnt_gt).astype(jnp.float32)
        eq_f = eq.astype(jnp.float32)
        tie_exc = _cumsum_2d(eq_f) - eq_f
        mask = jnp.logical_or(gt, jnp.logical_and(eq, tie_exc < r))
        cum = _cumsum_2d(mask.astype(jnp.float32))
        cum_ref[b * 64:(b + 1) * 64, :] = cum.astype(jnp.int32)


def _select(scores_2d):
    return pl.pallas_call(
        _select_body,
        in_specs=[pl.BlockSpec((256, 128), lambda: (0, 0))],
        out_specs=pl.BlockSpec((256, 128), lambda: (0, 0)),
        out_shape=jax.ShapeDtypeStruct((256, 128), jnp.int32),
    )(scores_2d)


def _sc_body(cum_hbm, tok_hbm, rows_out, idx_out,
             cum_v, idx_v, gidx_v, oidx_v, rows_v, sem):
    wid = lax.axis_index("s") * NC + lax.axis_index("c")
    b = wid // 8
    j = wid % 8
    pbase = j * PER_W
    rbase = jnp.where(j == 7, NUM_KEEP - PER_W, pbase)
    pltpu.sync_copy(cum_hbm.at[pl.ds(pl.multiple_of(b * N, 8), N)], cum_v)

    def make_search(base, out_ref, off):
        def search(ci, _):
            target = base + ci * 16 + lax.iota(jnp.int32, 16) + 1
            lb = jnp.zeros((16,), jnp.int32)
            for step in (4096, 2048, 1024, 512, 256, 128, 64, 32, 16, 8, 4,
                         2, 1):
                probe = lb + (step - 1)
                v = plsc.load_gather(cum_v, [probe])
                lb = lb + jnp.where(v < target, step, 0)
            out_ref[pl.ds(ci * 16, 16)] = lb + off
            return 0
        return search

    lax.fori_loop(0, SCHUNK, make_search(pbase, idx_v, 0), 0)
    pltpu.sync_copy(idx_v, idx_out.at[pl.ds(pl.multiple_of(wid * PER_W, 8),
                                            PER_W)])

    @pl.when(j < 7)
    def _():
        def cp(ci, _):
            gidx_v[pl.ds(ci * 16, 16)] = idx_v[pl.ds(ci * 16, 16)] + b * N
            return 0
        lax.fori_loop(0, SCHUNK, cp, 0)

    @pl.when(j == 7)
    def _():
        lax.fori_loop(0, SCHUNK,
                      make_search(NUM_KEEP - PER_W, gidx_v, b * N), 0)

    slab = rows_out.at[b]

    def gather(c, _):
        for t in range(GCHUNK // 16):
            oidx_v[c, pl.ds(t * 16, 16)] = (
                rbase + c * GCHUNK + t * 16 + lax.iota(jnp.int32, 16))
        idx_slice = gidx_v.at[pl.ds(c * GCHUNK, GCHUNK)]
        pltpu.async_copy(tok_hbm.at[idx_slice], rows_v, sem).wait()
        pltpu.async_copy(rows_v, slab.at[oidx_v.at[c]], sem).wait()
        return 0

    lax.fori_loop(0, NGC, gather, 0)


def _sc_gather(cum_flat, tokens_flat):
    mesh = plsc.VectorSubcoreMesh(core_axis_name="c", subcore_axis_name="s",
                                  num_cores=NC, num_subcores=NS)
    f = pl.kernel(
        _sc_body,
        out_type=(
            jax.ShapeDtypeStruct((B, NUM_KEEP, C), jnp.float32),
            jax.ShapeDtypeStruct((B * KPAD,), jnp.int32),
        ),
        mesh=mesh,
        compiler_params=pltpu.CompilerParams(needs_layout_passes=False),
        scratch_types=[
            pltpu.VMEM((N,), jnp.int32),
            pltpu.VMEM((PER_W,), jnp.int32),
            pltpu.VMEM((PER_W,), jnp.int32),
            pltpu.VMEM((NGC, GCHUNK), jnp.int32),
            pltpu.VMEM((GCHUNK, C), jnp.float32),
            pltpu.SemaphoreType.DMA,
        ],
    )
    return f(cum_flat, tokens_flat)


def kernel(tokens, spatial_shape, W1, b1, W2, b2):
    tokens_flat = tokens.reshape(B * N, C)
    scores = _scores(tokens_flat, W1, b1, W2, b2)
    cum = _select(scores.reshape(256, 128))
    rows, idx = _sc_gather(cum.reshape(B * N), tokens_flat)
    top_idx = idx.reshape(B, KPAD)[:, :NUM_KEEP]
    return rows, top_idx


# speedup vs baseline: 2.6502x; 1.0303x over previous
import functools

import jax
import jax.numpy as jnp
import numpy as np
from jax import lax
from jax.experimental import pallas as pl
from jax.experimental.pallas import tpu as pltpu
from jax.experimental.pallas import tpu_sc as plsc

B, N, C = 4, 8192, 768
HID = 128
NUM_KEEP = 5734

NC, NS = 2, 16
NW = NC * NS
KPAD = 5760
PER_W = (B * KPAD) // NW
GCHUNK = 48
NGC = PER_W // GCHUNK
SCHUNK = PER_W // 16

_MININT = np.int32(-2**31)
_BITVALS = [np.int32(np.uint32(1 << i)) for i in range(31, -1, -1)]


def _scores_body(tok_ref, w1t_ref, b1_ref, w2t_ref, b2_ref, out_ref):
    x = tok_ref[...]
    h = jnp.dot(x, w1t_ref[...], preferred_element_type=jnp.float32)
    h = h + b1_ref[...]
    h = 0.5 * h * (1.0 + lax.erf(h * np.float32(0.7071067811865476)))
    s = jnp.dot(h, w2t_ref[...], preferred_element_type=jnp.float32)
    out_ref[...] = s + b2_ref[...]


def _scores(tokens_flat, W1, b1, W2, b2):
    blk = 1024
    grid = (tokens_flat.shape[0] // blk,)
    return pl.pallas_call(
        _scores_body,
        grid=grid,
        in_specs=[
            pl.BlockSpec((blk, C), lambda i: (i, 0)),
            pl.BlockSpec((C, HID), lambda i: (0, 0)),
            pl.BlockSpec((1, HID), lambda i: (0, 0)),
            pl.BlockSpec((HID, 1), lambda i: (0, 0)),
            pl.BlockSpec((1, 1), lambda i: (0, 0)),
        ],
        out_specs=pl.BlockSpec((blk, 1), lambda i: (i, 0)),
        out_shape=jax.ShapeDtypeStruct((tokens_flat.shape[0], 1), jnp.float32),
    )(tokens_flat, W1.T, b1.reshape(1, HID), W2.T, b2.reshape(1, 1))


def _cumsum_2d(m):
    hi = lax.broadcasted_iota(jnp.int32, (128, 128), 0)
    wi = lax.broadcasted_iota(jnp.int32, (128, 128), 1)
    upper = (hi <= wi).astype(jnp.float32)
    incl_row = jnp.dot(m, upper, precision=lax.Precision.HIGHEST,
                       preferred_element_type=jnp.float32)
    rowtot = jnp.broadcast_to(incl_row[:, 127:128], (64, 128))
    ri = lax.broadcasted_iota(jnp.int32, (64, 64), 0)
    ci = lax.broadcasted_iota(jnp.int32, (64, 64), 1)
    strict = (ci < ri).astype(jnp.float32)
    rowoff = jnp.dot(strict, rowtot, precision=lax.Precision.HIGHEST,
                     preferred_element_type=jnp.float32)
    return incl_row + rowoff


def _select_body(s_ref, cum_ref):
    s = s_ref[...]
    bits = lax.bitcast_convert_type(s, jnp.int32)
    key = bits ^ (lax.shift_right_arithmetic(bits, 31) & np.int32(0x7FFFFFFF))
    for b in range(B):
        w = key[b * 64:(b + 1) * 64, :]
        t_u = jnp.int32(0)
        for bv in _BITVALS:
            cand_u = t_u | bv
            cand_s = cand_u ^ _MININT
            cnt = jnp.sum((w >= cand_s).astype(jnp.int32))
            t_u = jnp.where(cnt >= NUM_KEEP, cand_u, t_u)
        t_s = t_u ^ _MININT
        gt = (w > t_s)
        eq = (w == t_s)
        cnt_gt = jnp.sum(gt.astype(jnp.int32))
        r = (NUM_KEEP - cnt_gt).astype(jnp.float32)
        eq_f = eq.astype(jnp.float32)
        tie_exc = _cumsum_2d(eq_f) - eq_f
        mask = jnp.logical_or(gt, jnp.logical_and(eq, tie_exc < r))
        cum = _cumsum_2d(mask.astype(jnp.float32))
        cum_ref[b * 64:(b + 1) * 64, :] = cum.astype(jnp.int32)


def _select(scores_2d):
    return pl.pallas_call(
        _select_body,
        in_specs=[pl.BlockSpec((256, 128), lambda: (0, 0))],
        out_specs=pl.BlockSpec((256, 128), lambda: (0, 0)),
        out_shape=jax.ShapeDtypeStruct((256, 128), jnp.int32),
    )(scores_2d)


def _sc_body(cum_hbm, tok_hbm, rows_out, idx_out,
             cum_v, idx_v, gidx_v, oidx_v, rows_v, rows_v2,
             gsem, gsem2, ssem, ssem2):
    wid = lax.axis_index("s") * NC + lax.axis_index("c")
    b = wid // 8
    j = wid % 8
    pbase = j * PER_W
    rbase = jnp.where(j == 7, NUM_KEEP - PER_W, pbase)
    pltpu.sync_copy(cum_hbm.at[pl.ds(pl.multiple_of(b * N, 8), N)], cum_v)

    def make_search(base, out_ref, off):
        def search(ci, _):
            target = base + ci * 16 + lax.iota(jnp.int32, 16) + 1
            lb = jnp.zeros((16,), jnp.int32)
            for step in (4096, 2048, 1024, 512, 256, 128, 64, 32, 16, 8, 4,
                         2, 1):
                probe = lb + (step - 1)
                v = plsc.load_gather(cum_v, [probe])
                lb = lb + jnp.where(v < target, step, 0)
            out_ref[pl.ds(ci * 16, 16)] = lb + off
            return 0
        return search

    lax.fori_loop(0, SCHUNK, make_search(pbase, idx_v, 0), 0)
    pltpu.sync_copy(idx_v, idx_out.at[pl.ds(pl.multiple_of(wid * PER_W, 8),
                                            PER_W)])

    @pl.when(j < 7)
    def _():
        def cp(ci, _):
            gidx_v[pl.ds(ci * 16, 16)] = idx_v[pl.ds(ci * 16, 16)] + b * N
            return 0
        lax.fori_loop(0, SCHUNK, cp, 0)

    @pl.when(j == 7)
    def _():
        lax.fori_loop(0, SCHUNK,
                      make_search(NUM_KEEP - PER_W, gidx_v, b * N), 0)

    slab = rows_out.at[b]
    for c in range(NGC):
        for t in range(GCHUNK // 16):
            oidx_v[c, pl.ds(t * 16, 16)] = (
                rbase + c * GCHUNK + t * 16 + lax.iota(jnp.int32, 16))

    bufs = (rows_v, rows_v2)
    gsems = (gsem, gsem2)
    ssems = (ssem, ssem2)

    def fire_g(c):
        idx_slice = gidx_v.at[pl.ds(c * GCHUNK, GCHUNK)]
        return pltpu.async_copy(tok_hbm.at[idx_slice], bufs[c % 2],
                                gsems[c % 2])

    def fire_s(c):
        return pltpu.async_copy(bufs[c % 2], slab.at[oidx_v.at[c]],
                                ssems[c % 2])

    hg = [None] * NGC
    hs = [None] * NGC
    hg[0] = fire_g(0)
    for c in range(NGC):
        hg[c].wait()
        if c + 1 < NGC:
            if c >= 1:
                hs[c - 1].wait()
            hg[c + 1] = fire_g(c + 1)
        hs[c] = fire_s(c)
    hs[NGC - 2].wait()
    hs[NGC - 1].wait()


def _sc_gather(cum_flat, tokens_flat):
    mesh = plsc.VectorSubcoreMesh(core_axis_name="c", subcore_axis_name="s",
                                  num_cores=NC, num_subcores=NS)
    f = pl.kernel(
        _sc_body,
        out_type=(
            jax.ShapeDtypeStruct((B, NUM_KEEP, C), jnp.float32),
            jax.ShapeDtypeStruct((B * KPAD,), jnp.int32),
        ),
        mesh=mesh,
        compiler_params=pltpu.CompilerParams(needs_layout_passes=False),
        scratch_types=[
            pltpu.VMEM((N,), jnp.int32),
            pltpu.VMEM((PER_W,), jnp.int32),
            pltpu.VMEM((PER_W,), jnp.int32),
            pltpu.VMEM((NGC, GCHUNK), jnp.int32),
            pltpu.VMEM((GCHUNK, C), jnp.float32),
            pltpu.VMEM((GCHUNK, C), jnp.float32),
            pltpu.SemaphoreType.DMA,
            pltpu.SemaphoreType.DMA,
            pltpu.SemaphoreType.DMA,
            pltpu.SemaphoreType.DMA,
        ],
    )
    return f(cum_flat, tokens_flat)


def kernel(tokens, spatial_shape, W1, b1, W2, b2):
    tokens_flat = tokens.reshape(B * N, C)
    scores = _scores(tokens_flat, W1, b1, W2, b2)
    cum = _select(scores.reshape(256, 128))
    rows, idx = _sc_gather(cum.reshape(B * N), tokens_flat)
    top_idx = idx.reshape(B, KPAD)[:, :NUM_KEEP]
    return rows, top_idx


# speedup vs baseline: 2.9360x; 1.1078x over previous
import functools

import jax
import jax.numpy as jnp
import numpy as np
from jax import lax
from jax.experimental import pallas as pl
from jax.experimental.pallas import tpu as pltpu
from jax.experimental.pallas import tpu_sc as plsc

B, N, C = 4, 8192, 768
HID = 128
NUM_KEEP = 5734

NC, NS = 2, 16
NW = NC * NS
KPAD = 5760
PER_W = (B * KPAD) // NW
GCHUNK = 48
NGC = PER_W // GCHUNK
SCHUNK = PER_W // 16

_MININT = np.int32(-2**31)
_BITVALS = [np.int32(np.uint32(1 << i)) for i in range(31, -1, -1)]


def _scores_body(tok_ref, w1t_ref, b1_ref, w2t_ref, b2_ref, out_ref):
    x = tok_ref[...]
    h = jnp.dot(x, w1t_ref[...], preferred_element_type=jnp.float32)
    h = h + b1_ref[...]
    h = 0.5 * h * (1.0 + lax.erf(h * np.float32(0.7071067811865476)))
    s = jnp.dot(h, w2t_ref[...], preferred_element_type=jnp.float32)
    s = s + b2_ref[...]
    out_ref[...] = s.reshape(out_ref.shape)


def _scores(tokens_flat, W1, b1, W2, b2):
    blk = 4096
    grid = (tokens_flat.shape[0] // blk,)
    return pl.pallas_call(
        _scores_body,
        grid=grid,
        in_specs=[
            pl.BlockSpec((blk, C), lambda i: (i, 0)),
            pl.BlockSpec((C, HID), lambda i: (0, 0)),
            pl.BlockSpec((1, HID), lambda i: (0, 0)),
            pl.BlockSpec((HID, 1), lambda i: (0, 0)),
            pl.BlockSpec((1, 1), lambda i: (0, 0)),
        ],
        out_specs=pl.BlockSpec((blk // 128, 128), lambda i: (i, 0)),
        out_shape=jax.ShapeDtypeStruct((256, 128), jnp.float32),
    )(tokens_flat, W1.T, b1.reshape(1, HID), W2.T, b2.reshape(1, 1))


def _cumsum_2d(m):
    hi = lax.broadcasted_iota(jnp.int32, (128, 128), 0)
    wi = lax.broadcasted_iota(jnp.int32, (128, 128), 1)
    upper = (hi <= wi).astype(jnp.float32)
    incl_row = jnp.dot(m, upper, precision=lax.Precision.HIGHEST,
                       preferred_element_type=jnp.float32)
    rowtot = jnp.broadcast_to(incl_row[:, 127:128], (64, 128))
    ri = lax.broadcasted_iota(jnp.int32, (64, 64), 0)
    ci = lax.broadcasted_iota(jnp.int32, (64, 64), 1)
    strict = (ci < ri).astype(jnp.float32)
    rowoff = jnp.dot(strict, rowtot, precision=lax.Precision.HIGHEST,
                     preferred_element_type=jnp.float32)
    return incl_row + rowoff


def _select_body(s_ref, cum_ref):
    s = s_ref[...]
    bits = lax.bitcast_convert_type(s, jnp.int32)
    key = bits ^ (lax.shift_right_arithmetic(bits, 31) & np.int32(0x7FFFFFFF))
    for b in range(B):
        w = key[b * 64:(b + 1) * 64, :]
        t_u = jnp.int32(0)
        for bv in _BITVALS:
            cand_u = t_u | bv
            cand_s = cand_u ^ _MININT
            cnt = jnp.sum((w >= cand_s).astype(jnp.int32))
            t_u = jnp.where(cnt >= NUM_KEEP, cand_u, t_u)
        t_s = t_u ^ _MININT
        gt = (w > t_s)
        eq = (w == t_s)
        cnt_gt = jnp.sum(gt.astype(jnp.int32))
        r = (NUM_KEEP - cnt_gt).astype(jnp.float32)
        eq_f = eq.astype(jnp.float32)
        tie_exc = _cumsum_2d(eq_f) - eq_f
        mask = jnp.logical_or(gt, jnp.logical_and(eq, tie_exc < r))
        cum = _cumsum_2d(mask.astype(jnp.float32))
        cum_ref[b * 64:(b + 1) * 64, :] = cum.astype(jnp.int32)


def _select(scores_2d):
    return pl.pallas_call(
        _select_body,
        in_specs=[pl.BlockSpec((256, 128), lambda: (0, 0))],
        out_specs=pl.BlockSpec((256, 128), lambda: (0, 0)),
        out_shape=jax.ShapeDtypeStruct((256, 128), jnp.int32),
    )(scores_2d)


def _sc_body(cum_hbm, tok_hbm, rows_out, idx_out,
             cum_v, idx_v, gidx_v, oidx_v, rows_v, rows_v2,
             gsem, gsem2, ssem, ssem2):
    wid = lax.axis_index("s") * NC + lax.axis_index("c")
    b = wid // 8
    j = wid % 8
    pbase = j * PER_W
    rbase = jnp.where(j == 7, NUM_KEEP - PER_W, pbase)
    pltpu.sync_copy(cum_hbm.at[pl.ds(pl.multiple_of(b * N, 8), N)], cum_v)

    def make_search(base, out_ref, off):
        def search(ci, _):
            target = base + ci * 16 + lax.iota(jnp.int32, 16) + 1
            lb = jnp.zeros((16,), jnp.int32)
            for step in (4096, 2048, 1024, 512, 256, 128, 64, 32, 16, 8, 4,
                         2, 1):
                probe = lb + (step - 1)
                v = plsc.load_gather(cum_v, [probe])
                lb = lb + jnp.where(v < target, step, 0)
            out_ref[pl.ds(ci * 16, 16)] = lb + off
            return 0
        return search

    lax.fori_loop(0, SCHUNK, make_search(pbase, idx_v, 0), 0)
    pltpu.sync_copy(idx_v, idx_out.at[pl.ds(pl.multiple_of(wid * PER_W, 8),
                                            PER_W)])

    @pl.when(j < 7)
    def _():
        def cp(ci, _):
            gidx_v[pl.ds(ci * 16, 16)] = idx_v[pl.ds(ci * 16, 16)] + b * N
            return 0
        lax.fori_loop(0, SCHUNK, cp, 0)

    @pl.when(j == 7)
    def _():
        lax.fori_loop(0, SCHUNK,
                      make_search(NUM_KEEP - PER_W, gidx_v, b * N), 0)

    slab = rows_out.at[b]
    for c in range(NGC):
        for t in range(GCHUNK // 16):
            oidx_v[c, pl.ds(t * 16, 16)] = (
                rbase + c * GCHUNK + t * 16 + lax.iota(jnp.int32, 16))

    bufs = (rows_v, rows_v2)
    gsems = (gsem, gsem2)
    ssems = (ssem, ssem2)

    def fire_g(c):
        idx_slice = gidx_v.at[pl.ds(c * GCHUNK, GCHUNK)]
        return pltpu.async_copy(tok_hbm.at[idx_slice], bufs[c % 2],
                                gsems[c % 2])

    def fire_s(c):
        return pltpu.async_copy(bufs[c % 2], slab.at[oidx_v.at[c]],
                                ssems[c % 2])

    hg = [None] * NGC
    hs = [None] * NGC
    hg[0] = fire_g(0)
    for c in range(NGC):
        hg[c].wait()
        if c + 1 < NGC:
            if c >= 1:
                hs[c - 1].wait()
            hg[c + 1] = fire_g(c + 1)
        hs[c] = fire_s(c)
    hs[NGC - 2].wait()
    hs[NGC - 1].wait()


def _sc_gather(cum_flat, tokens_flat):
    mesh = plsc.VectorSubcoreMesh(core_axis_name="c", subcore_axis_name="s",
                                  num_cores=NC, num_subcores=NS)
    f = pl.kernel(
        _sc_body,
        out_type=(
            jax.ShapeDtypeStruct((B, NUM_KEEP, C), jnp.float32),
            jax.ShapeDtypeStruct((B * KPAD,), jnp.int32),
        ),
        mesh=mesh,
        compiler_params=pltpu.CompilerParams(needs_layout_passes=False),
        scratch_types=[
            pltpu.VMEM((N,), jnp.int32),
            pltpu.VMEM((PER_W,), jnp.int32),
            pltpu.VMEM((PER_W,), jnp.int32),
            pltpu.VMEM((NGC, GCHUNK), jnp.int32),
            pltpu.VMEM((GCHUNK, C), jnp.float32),
            pltpu.VMEM((GCHUNK, C), jnp.float32),
            pltpu.SemaphoreType.DMA,
            pltpu.SemaphoreType.DMA,
            pltpu.SemaphoreType.DMA,
            pltpu.SemaphoreType.DMA,
        ],
    )
    return f(cum_flat, tokens_flat)


def kernel(tokens, spatial_shape, W1, b1, W2, b2):
    tokens_flat = tokens.reshape(B * N, C)
    scores = _scores(tokens_flat, W1, b1, W2, b2)
    cum = _select(scores)
    rows, idx = _sc_gather(cum.reshape(B * N), tokens_flat)
    top_idx = idx.reshape(B, KPAD)[:, :NUM_KEEP]
    return rows, top_idx
